# baseline (speedup 1.0000x reference)
import functools

import jax
import jax.numpy as jnp
from jax import lax
from jax.experimental import pallas as pl
from jax.experimental.pallas import tpu as pltpu
from jax.experimental.pallas import tpu_sc as plsc

NC = 2
NS = 16
NW = NC * NS
CHW = 128
CA = 80
PAD_N = 16


def _sc_mesh():
    return plsc.VectorSubcoreMesh(core_axis_name="c", subcore_axis_name="s",
                                  num_cores=NC, num_subcores=NS)


def _row_split(N):
    rpt = (N // NS) // 8 * 8
    tail = N - NS * rpt
    return rpt, tail


def _copy_rows(copy_one, s, N):
    rpt, tail = _row_split(N)
    copy_one(s * rpt, rpt)
    if tail:
        @pl.when(s == NS - 1)
        def _():
            copy_one(NS * rpt, tail)


def _fill_chunk(dst_small, src_big, base, n, clamp=None):
    for k in range(n // 16):
        v = src_big[pl.ds(base + k * 16, 16)]
        if clamp is not None:
            v = jnp.minimum(v, clamp)
        dst_small[pl.ds(k * 16, 16)] = v


def _make_deg_kernel(rpe, N):
    Np = N + PAD_N
    ept = rpe * CHW
    rpt, _ = _row_split(N)

    @functools.partial(
        pl.kernel,
        out_type=[
            jax.ShapeDtypeStruct((N,), jnp.float32),
            jax.ShapeDtypeStruct((N,), jnp.float32),
            jax.ShapeDtypeStruct((N,), jnp.float32),
            jax.ShapeDtypeStruct((N,), jnp.float32),
        ],
        mesh=_sc_mesh(),
        scratch_types=[
            pltpu.VMEM((ept,), jnp.int32),
            pltpu.VMEM((ept,), jnp.int32),
            pltpu.VMEM((CHW,), jnp.int32),
            pltpu.VMEM((CHW,), jnp.int32),
            pltpu.VMEM((CHW,), jnp.float32),
            pltpu.VMEM((rpt,), jnp.float32),
            pltpu.VMEM_SHARED((Np,), jnp.float32),
            pltpu.VMEM_SHARED((Np,), jnp.float32),
        ],
    )
    def deg_kernel(src_hbm, dst_hbm,
                   degs0_out, degs1_out, degd0_out, degd1_out,
                   sidxb, didxb, six, dix, ones_v, stage, degs_sh, degd_sh):
        c = lax.axis_index("c")
        s = lax.axis_index("s")
        wid = s * NC + c

        pltpu.sync_copy(src_hbm.at[pl.ds(wid * ept, ept)], sidxb)
        pltpu.sync_copy(dst_hbm.at[pl.ds(wid * ept, ept)], didxb)
        for i in range(CHW // 16):
            ones_v[pl.ds(i * 16, 16)] = jnp.full((16,), 1.0, jnp.float32)
        for i in range(rpt // 16):
            stage[pl.ds(i * 16, 16)] = jnp.full((16,), 0.0, jnp.float32)

        def zero_rows(b, r):
            pltpu.sync_copy(stage.at[pl.ds(0, r)], degs_sh.at[pl.ds(b, r)])
            pltpu.sync_copy(stage.at[pl.ds(0, r)], degd_sh.at[pl.ds(b, r)])

        _copy_rows(zero_rows, s, N)

        @pl.when(s == 0)
        def _():
            zero_rows(N, PAD_N)

        plsc.subcore_barrier()

        def step(j, carry):
            _fill_chunk(six, sidxb, j * CHW, CHW)
            _fill_chunk(dix, didxb, j * CHW, CHW)
            pltpu.sync_copy(ones_v, degs_sh.at[six], add=True)
            pltpu.sync_copy(ones_v, degd_sh.at[dix], add=True)
            return carry

        lax.fori_loop(0, rpe, step, 0)
        plsc.subcore_barrier()

        def copy_via_stage(tab_sh, out_hbm, b, r):
            pltpu.sync_copy(tab_sh.at[pl.ds(b, r)], stage.at[pl.ds(0, r)])
            pltpu.sync_copy(stage.at[pl.ds(0, r)], out_hbm.at[pl.ds(b, r)])

        def out_rows(b, r):
            @pl.when(c == 0)
            def _():
                copy_via_stage(degs_sh, degs0_out, b, r)
                copy_via_stage(degd_sh, degd0_out, b, r)

            @pl.when(c == 1)
            def _():
                copy_via_stage(degs_sh, degs1_out, b, r)
                copy_via_stage(degd_sh, degd1_out, b, r)

        _copy_rows(out_rows, s, N)

    return deg_kernel


def _make_agg_kernel(n0, n1, N, H):
    nmax = max(n0, n1)

    @functools.partial(
        pl.kernel,
        out_type=jax.ShapeDtypeStruct((NC, N, H), jnp.float32),
        mesh=_sc_mesh(),
        scratch_types=[
            pltpu.VMEM((nmax * CA,), jnp.int32),
            pltpu.VMEM((nmax * CA,), jnp.int32),
            pltpu.VMEM((CA,), jnp.int32),
            pltpu.VMEM((CA,), jnp.int32),
            pltpu.VMEM((CA, H), jnp.float32),
            pltpu.VMEM((CA, H), jnp.float32),
            pltpu.VMEM_SHARED((N, H), jnp.float32),
            pltpu.SemaphoreType.DMA,
            pltpu.SemaphoreType.DMA,
        ],
    )
    def agg_kernel(src_hbm, dst_hbm, hn_hbm, zagg_hbm,
                   agg_out,
                   sidxb, didxb, dix0, dix1, rows0, rows1, agg_sh, sem0, sem1):
        c = lax.axis_index("c")
        s = lax.axis_index("s")

        def zero_rows(b, r):
            pltpu.sync_copy(zagg_hbm.at[pl.ds(b, r)], agg_sh.at[pl.ds(b, r)])

        _copy_rows(zero_rows, s, N)

        def gather(i, buf, sem):
            return pltpu.async_copy(
                hn_hbm.at[sidxb.at[pl.ds(i * CA, CA)]], buf, sem)

        def gwait(i, buf, sem):
            pltpu.make_async_copy(
                hn_hbm.at[sidxb.at[pl.ds(i * CA, CA)]], buf, sem).wait()

        def do_chunks(nch, gbase):
            ept = nch * CA
            pltpu.sync_copy(src_hbm.at[pl.ds(gbase * CA, ept)],
                            sidxb.at[pl.ds(0, ept)])
            pltpu.sync_copy(dst_hbm.at[pl.ds(gbase * CA, ept)],
                            didxb.at[pl.ds(0, ept)])
            plsc.subcore_barrier()
            gather(0, rows0, sem0)

            def pair(j, carry):
                i0 = j * 2
                gather(i0 + 1, rows1, sem1)
                _fill_chunk(dix0, didxb, i0 * CA, CA, clamp=N - 1)
                gwait(i0, rows0, sem0)
                pltpu.sync_copy(rows0, agg_sh.at[dix0], add=True)

                @pl.when(j < nch // 2 - 1)
                def _():
                    gather(i0 + 2, rows0, sem0)

                _fill_chunk(dix1, didxb, (i0 + 1) * CA, CA, clamp=N - 1)
                gwait(i0 + 1, rows1, sem1)
                pltpu.sync_copy(rows1, agg_sh.at[dix1], add=True)
                return carry

            lax.fori_loop(0, nch // 2, pair, 0)

        @pl.when(c == 0)
        def _():
            do_chunks(n0, s * n0)

        @pl.when(c == 1)
        def _():
            do_chunks(n1, NS * n0 + s * n1)

        plsc.subcore_barrier()

        def out_rows(b, r):
            pltpu.sync_copy(agg_sh.at[pl.ds(b, r)], agg_out.at[c, pl.ds(b, r)])

        _copy_rows(out_rows, s, N)

    return agg_kernel


def _matmul_kernel(x_ref, w_ref, h_ref):
    h_ref[...] = jnp.dot(x_ref[...], w_ref[...],
                         preferred_element_type=jnp.float32)


def _scale_kernel(h_ref, deg0_ref, deg1_ref, hn_ref):
    deg = deg0_ref[...] + deg1_ref[...]
    norm = lax.rsqrt(jnp.maximum(deg, 1.0))
    hn_ref[...] = h_ref[...] * norm


def _finalize_kernel(aggp_ref, deg0_ref, deg1_ref, b_ref, out_ref):
    agg = aggp_ref[0] + aggp_ref[1]
    deg = deg0_ref[...] + deg1_ref[...]
    norm = lax.rsqrt(jnp.maximum(deg, 1.0))
    out_ref[...] = jnp.maximum(agg * norm + b_ref[...], 0.0)


def kernel(edge_index, x, W1, b1):
    N, D = x.shape
    H = W1.shape[1]
    E = edge_index.shape[1]

    rpe = -(-E // (NW * 8 * CHW)) * 8
    e_pad = NW * rpe * CHW
    pad = e_pad - E
    src = jnp.concatenate([edge_index[0], jnp.full((pad,), N, jnp.int32)])
    dst = jnp.concatenate([edge_index[1], jnp.full((pad,), N, jnp.int32)])

    degs0, degs1, degd0, degd1 = _make_deg_kernel(rpe, N)(src, dst)
    degs0 = degs0.reshape(N, 1)
    degs1 = degs1.reshape(N, 1)
    degd0 = degd0.reshape(N, 1)
    degd1 = degd1.reshape(N, 1)

    RB = 1000
    grid = (N // RB,)
    h = pl.pallas_call(
        _matmul_kernel,
        grid=grid,
        in_specs=[
            pl.BlockSpec((RB, D), lambda i: (i, 0)),
            pl.BlockSpec((D, H), lambda i: (0, 0)),
        ],
        out_specs=pl.BlockSpec((RB, H), lambda i: (i, 0)),
        out_shape=jax.ShapeDtypeStruct((N, H), jnp.float32),
    )(x, W1)
    hn = pl.pallas_call(
        _scale_kernel,
        grid=grid,
        in_specs=[
            pl.BlockSpec((RB, H), lambda i: (i, 0)),
            pl.BlockSpec((RB, 1), lambda i: (i, 0)),
            pl.BlockSpec((RB, 1), lambda i: (i, 0)),
        ],
        out_specs=pl.BlockSpec((RB, H), lambda i: (i, 0)),
        out_shape=jax.ShapeDtypeStruct((N, H), jnp.float32),
    )(h, degs0, degs1)

    ncht = e_pad // CA
    per_sc = ncht // NS
    n0 = min(per_sc - 2, max(2, round(per_sc * 0.5 / 2) * 2))
    n1 = per_sc - n0

    hn_pad = jnp.concatenate([hn, jnp.zeros((PAD_N, H), jnp.float32)])
    zagg = jnp.zeros((N, H), jnp.float32)
    aggp = _make_agg_kernel(n0, n1, N, H)(src, dst, hn_pad, zagg)

    out = pl.pallas_call(
        _finalize_kernel,
        grid=grid,
        in_specs=[
            pl.BlockSpec((NC, RB, H), lambda i: (0, i, 0)),
            pl.BlockSpec((RB, 1), lambda i: (i, 0)),
            pl.BlockSpec((RB, 1), lambda i: (i, 0)),
            pl.BlockSpec((1, H), lambda i: (0, 0)),
        ],
        out_specs=pl.BlockSpec((RB, H), lambda i: (i, 0)),
        out_shape=jax.ShapeDtypeStruct((N, H), jnp.float32),
    )(aggp, degd0, degd1, b1.reshape(1, H))
    return out


# speedup vs baseline: 1.1186x; 1.1186x over previous
import functools

import jax
import jax.numpy as jnp
from jax import lax
from jax.experimental import pallas as pl
from jax.experimental.pallas import tpu as pltpu
from jax.experimental.pallas import tpu_sc as plsc

NC = 2
NS = 16
NW = NC * NS
CHW = 128
CA = 80
PAD_N = 16


def _sc_mesh():
    return plsc.VectorSubcoreMesh(core_axis_name="c", subcore_axis_name="s",
                                  num_cores=NC, num_subcores=NS)


def _row_split(N):
    rpt = (N // NS) // 8 * 8
    tail = N - NS * rpt
    return rpt, tail


def _copy_rows(copy_one, s, N):
    rpt, tail = _row_split(N)
    copy_one(s * rpt, rpt)
    if tail:
        @pl.when(s == NS - 1)
        def _():
            copy_one(NS * rpt, tail)


def _fill_chunk(dst_small, src_big, base, n, clamp=None):
    for k in range(n // 16):
        v = src_big[pl.ds(base + k * 16, 16)]
        if clamp is not None:
            v = jnp.minimum(v, clamp)
        dst_small[pl.ds(k * 16, 16)] = v


def _make_deg_kernel(rpe, N):
    Np = N + PAD_N
    ept = rpe * CHW
    rpt, _ = _row_split(N)

    @functools.partial(
        pl.kernel,
        out_type=[
            jax.ShapeDtypeStruct((N,), jnp.float32),
            jax.ShapeDtypeStruct((N,), jnp.float32),
            jax.ShapeDtypeStruct((N,), jnp.float32),
            jax.ShapeDtypeStruct((N,), jnp.float32),
        ],
        mesh=_sc_mesh(),
        scratch_types=[
            pltpu.VMEM((ept,), jnp.int32),
            pltpu.VMEM((ept,), jnp.int32),
            pltpu.VMEM((CHW,), jnp.int32),
            pltpu.VMEM((CHW,), jnp.int32),
            pltpu.VMEM((CHW,), jnp.float32),
            pltpu.VMEM((rpt,), jnp.float32),
            pltpu.VMEM_SHARED((Np,), jnp.float32),
            pltpu.VMEM_SHARED((Np,), jnp.float32),
        ],
    )
    def deg_kernel(src_hbm, dst_hbm,
                   degs0_out, degs1_out, degd0_out, degd1_out,
                   sidxb, didxb, six, dix, ones_v, stage, degs_sh, degd_sh):
        c = lax.axis_index("c")
        s = lax.axis_index("s")
        wid = s * NC + c

        pltpu.sync_copy(src_hbm.at[pl.ds(wid * ept, ept)], sidxb)
        pltpu.sync_copy(dst_hbm.at[pl.ds(wid * ept, ept)], didxb)
        for i in range(CHW // 16):
            ones_v[pl.ds(i * 16, 16)] = jnp.full((16,), 1.0, jnp.float32)
        for i in range(rpt // 16):
            stage[pl.ds(i * 16, 16)] = jnp.full((16,), 0.0, jnp.float32)

        def zero_rows(b, r):
            pltpu.sync_copy(stage.at[pl.ds(0, r)], degs_sh.at[pl.ds(b, r)])
            pltpu.sync_copy(stage.at[pl.ds(0, r)], degd_sh.at[pl.ds(b, r)])

        _copy_rows(zero_rows, s, N)

        @pl.when(s == 0)
        def _():
            zero_rows(N, PAD_N)

        plsc.subcore_barrier()

        def step(j, carry):
            _fill_chunk(six, sidxb, j * CHW, CHW)
            _fill_chunk(dix, didxb, j * CHW, CHW)
            pltpu.sync_copy(ones_v, degs_sh.at[six], add=True)
            pltpu.sync_copy(ones_v, degd_sh.at[dix], add=True)
            return carry

        lax.fori_loop(0, rpe, step, 0)
        plsc.subcore_barrier()

        def copy_via_stage(tab_sh, out_hbm, b, r):
            pltpu.sync_copy(tab_sh.at[pl.ds(b, r)], stage.at[pl.ds(0, r)])
            pltpu.sync_copy(stage.at[pl.ds(0, r)], out_hbm.at[pl.ds(b, r)])

        def out_rows(b, r):
            @pl.when(c == 0)
            def _():
                copy_via_stage(degs_sh, degs0_out, b, r)
                copy_via_stage(degd_sh, degd0_out, b, r)

            @pl.when(c == 1)
            def _():
                copy_via_stage(degs_sh, degs1_out, b, r)
                copy_via_stage(degd_sh, degd1_out, b, r)

        _copy_rows(out_rows, s, N)

    return deg_kernel


def _make_agg_kernel(rpe, N, H):
    ept = rpe * CHW
    nch = ept // CA

    @functools.partial(
        pl.kernel,
        out_type=jax.ShapeDtypeStruct((NC, N, H), jnp.float32),
        mesh=_sc_mesh(),
        scratch_types=[
            pltpu.VMEM((ept,), jnp.int32),
            pltpu.VMEM((ept,), jnp.int32),
            pltpu.VMEM((CA,), jnp.int32),
            pltpu.VMEM((CA,), jnp.int32),
            pltpu.VMEM((CA, H), jnp.float32),
            pltpu.VMEM((CA, H), jnp.float32),
            pltpu.VMEM_SHARED((N, H), jnp.float32),
            pltpu.SemaphoreType.DMA,
            pltpu.SemaphoreType.DMA,
        ],
    )
    def agg_kernel(src_hbm, dst_hbm, hn_hbm, zagg_hbm,
                   agg_out,
                   sidxb, didxb, dix0, dix1, rows0, rows1, agg_sh, sem0, sem1):
        c = lax.axis_index("c")
        s = lax.axis_index("s")
        wid = s * NC + c

        pltpu.sync_copy(src_hbm.at[pl.ds(wid * ept, ept)], sidxb)
        pltpu.sync_copy(dst_hbm.at[pl.ds(wid * ept, ept)], didxb)

        def zero_rows(b, r):
            pltpu.sync_copy(zagg_hbm.at[pl.ds(b, r)], agg_sh.at[pl.ds(b, r)])

        _copy_rows(zero_rows, s, N)
        plsc.subcore_barrier()

        def gather(i, buf, sem):
            return pltpu.async_copy(
                hn_hbm.at[sidxb.at[pl.ds(i * CA, CA)]], buf, sem)

        def gwait(i, buf, sem):
            pltpu.make_async_copy(
                hn_hbm.at[sidxb.at[pl.ds(i * CA, CA)]], buf, sem).wait()

        gather(0, rows0, sem0)

        def pair(j, carry):
            i0 = j * 2
            gather(i0 + 1, rows1, sem1)
            _fill_chunk(dix0, didxb, i0 * CA, CA, clamp=N - 1)
            gwait(i0, rows0, sem0)
            pltpu.sync_copy(rows0, agg_sh.at[dix0], add=True)

            @pl.when(j < nch // 2 - 1)
            def _():
                gather(i0 + 2, rows0, sem0)

            _fill_chunk(dix1, didxb, (i0 + 1) * CA, CA, clamp=N - 1)
            gwait(i0 + 1, rows1, sem1)
            pltpu.sync_copy(rows1, agg_sh.at[dix1], add=True)
            return carry

        lax.fori_loop(0, nch // 2, pair, 0)
        plsc.subcore_barrier()

        def out_rows(b, r):
            pltpu.sync_copy(agg_sh.at[pl.ds(b, r)], agg_out.at[c, pl.ds(b, r)])

        _copy_rows(out_rows, s, N)

    return agg_kernel


def _matmul_norm_kernel(x_ref, w_ref, deg0_ref, deg1_ref, hn_ref):
    deg = deg0_ref[...] + deg1_ref[...]
    norm = lax.rsqrt(jnp.maximum(deg, 1.0))
    h = jnp.dot(x_ref[...], w_ref[...], preferred_element_type=jnp.float32)
    hn_ref[...] = h * norm


def _finalize_kernel(aggp_ref, deg0_ref, deg1_ref, b_ref, out_ref):
    agg = aggp_ref[0] + aggp_ref[1]
    deg = deg0_ref[...] + deg1_ref[...]
    norm = lax.rsqrt(jnp.maximum(deg, 1.0))
    out_ref[...] = jnp.maximum(agg * norm + b_ref[...], 0.0)


def kernel(edge_index, x, W1, b1):
    N, D = x.shape
    H = W1.shape[1]
    E = edge_index.shape[1]

    rpe = -(-E // (NW * 8 * CHW)) * 8
    e_pad = NW * rpe * CHW
    pad = e_pad - E
    src = jnp.concatenate([edge_index[0], jnp.full((pad,), N, jnp.int32)])
    dst = jnp.concatenate([edge_index[1], jnp.full((pad,), N, jnp.int32)])

    degs0, degs1, degd0, degd1 = _make_deg_kernel(rpe, N)(src, dst)
    degs0 = degs0.reshape(N, 1)
    degs1 = degs1.reshape(N, 1)
    degd0 = degd0.reshape(N, 1)
    degd1 = degd1.reshape(N, 1)

    RB = 1000
    grid = (N // RB,)
    hn = pl.pallas_call(
        _matmul_norm_kernel,
        grid=grid,
        in_specs=[
            pl.BlockSpec((RB, D), lambda i: (i, 0)),
            pl.BlockSpec((D, H), lambda i: (0, 0)),
            pl.BlockSpec((RB, 1), lambda i: (i, 0)),
            pl.BlockSpec((RB, 1), lambda i: (i, 0)),
        ],
        out_specs=pl.BlockSpec((RB, H), lambda i: (i, 0)),
        out_shape=jax.ShapeDtypeStruct((N, H), jnp.float32),
    )(x, W1, degs0, degs1)

    hn_pad = jnp.concatenate([hn, jnp.zeros((PAD_N, H), jnp.float32)])
    zagg = jnp.zeros((N, H), jnp.float32)
    aggp = _make_agg_kernel(rpe, N, H)(src, dst, hn_pad, zagg)

    out = pl.pallas_call(
        _finalize_kernel,
        grid=grid,
        in_specs=[
            pl.BlockSpec((NC, RB, H), lambda i: (0, i, 0)),
            pl.BlockSpec((RB, 1), lambda i: (i, 0)),
            pl.BlockSpec((RB, 1), lambda i: (i, 0)),
            pl.BlockSpec((1, H), lambda i: (0, 0)),
        ],
        out_specs=pl.BlockSpec((RB, H), lambda i: (i, 0)),
        out_shape=jax.ShapeDtypeStruct((N, H), jnp.float32),
    )(aggp, degd0, degd1, b1.reshape(1, H))
    return out
